# Initial kernel scaffold; baseline (speedup 1.0000x reference)
import jax
import jax.numpy as jnp
from jax.experimental import pallas as pl


def kernel(adj_indices, adj_values, embs):
    raise NotImplementedError("write your pallas kernel here")


# speedup vs baseline: 15.0842x; 15.0842x over previous
import jax
import jax.numpy as jnp
from jax import lax
from jax.experimental import pallas as pl
from jax.experimental.pallas import tpu as pltpu
from jax.experimental.pallas import tpu_sc as plsc

N = 16384
D = 64
DH = 32
NNZ = 2684354
LEAKY = 0.2

NW = 32
CHUNK = 1024
KSTREAM = CHUNK // 128
CHUNKS_PER_W = -(-NNZ // (NW * CHUNK))
NNZ_PAD = NW * CHUNK * CHUNKS_PER_W
ROWS_PER_SUB = N // 16


def _propagate_body(src_hbm, dst_hbm, vals_hbm, table_hbm, out_hbm,
                    src_v, dst_v, vals_v, rowbuf, acc, sem):
    cid = lax.axis_index("c")
    sid = lax.axis_index("s")
    wid = sid * 2 + cid

    @pl.loop(0, ROWS_PER_SUB)
    def _zero(i):
        for k in range(DH // 16):
            rowbuf[i, pl.ds(k * 16, 16)] = jnp.zeros((16,), jnp.float32)

    pltpu.sync_copy(rowbuf.at[pl.ds(0, ROWS_PER_SUB)],
                    acc.at[pl.ds(sid * ROWS_PER_SUB, ROWS_PER_SUB)])
    plsc.subcore_barrier()

    @pl.loop(0, CHUNKS_PER_W)
    def _chunk(g):
        row0 = (wid * CHUNKS_PER_W + g) * KSTREAM
        e0 = row0 * 128
        pltpu.sync_copy(src_hbm.at[pl.ds(row0, KSTREAM)], src_v)
        pltpu.sync_copy(dst_hbm.at[pl.ds(row0, KSTREAM)], dst_v)
        pltpu.sync_copy(vals_hbm.at[pl.ds(e0, CHUNK)], vals_v)

        descs = []
        for j in range(KSTREAM):
            descs.append(pltpu.async_copy(
                table_hbm.at[src_v.at[j]],
                rowbuf.at[pl.ds(j * 128, 128)], sem))
        for d in descs:
            d.wait()

        @pl.loop(0, CHUNK // 16)
        def _scale(i16):
            vv = vals_v[pl.ds(i16 * 16, 16)]
            for lane in range(16):
                v = vv[lane]
                e = i16 * 16 + lane
                for k in range(DH // 16):
                    sl = pl.ds(k * 16, 16)
                    rowbuf[e, sl] = rowbuf[e, sl] * v

        for j in range(KSTREAM):
            pltpu.sync_copy(rowbuf.at[pl.ds(j * 128, 128)],
                            acc.at[dst_v.at[j]], add=True)

    plsc.subcore_barrier()
    pltpu.sync_copy(acc.at[pl.ds(sid * ROWS_PER_SUB, ROWS_PER_SUB)],
                    out_hbm.at[pl.ds(cid * N + sid * ROWS_PER_SUB,
                                     ROWS_PER_SUB)])


_propagate = pl.kernel(
    _propagate_body,
    out_type=jax.ShapeDtypeStruct((2 * N, DH), jnp.float32),
    mesh=plsc.VectorSubcoreMesh(core_axis_name="c", subcore_axis_name="s"),
    scratch_types=[
        pltpu.VMEM((KSTREAM, 128), jnp.int32),
        pltpu.VMEM((KSTREAM, 128), jnp.int32),
        pltpu.VMEM((CHUNK,), jnp.float32),
        pltpu.VMEM((CHUNK, DH), jnp.float32),
        pltpu.VMEM_SHARED((N, DH), jnp.float32),
        pltpu.SemaphoreType.DMA,
    ],
    compiler_params=pltpu.CompilerParams(use_tc_tiling_on_sc=False),
)


def _combine_add_body(p_ref, o_ref):
    o_ref[...] = p_ref[0] + p_ref[1]


_combine_add = pl.pallas_call(
    _combine_add_body,
    grid=(N // 2048,),
    in_specs=[pl.BlockSpec((2, 2048, DH), lambda i: (0, i, 0))],
    out_specs=pl.BlockSpec((2048, DH), lambda i: (i, 0)),
    out_shape=jax.ShapeDtypeStruct((N, DH), jnp.float32),
)


def _combine_act_body(h0_ref, h1_ref, o_ref):
    z0 = h0_ref[0] + h0_ref[1]
    z1 = h1_ref[0] + h1_ref[1]
    z = jnp.concatenate([z0, z1], axis=1)
    o_ref[...] = jnp.where(z >= 0, z, LEAKY * z)


_combine_act = pl.pallas_call(
    _combine_act_body,
    grid=(N // 2048,),
    in_specs=[pl.BlockSpec((2, 2048, DH), lambda i: (0, i, 0)),
              pl.BlockSpec((2, 2048, DH), lambda i: (0, i, 0))],
    out_specs=pl.BlockSpec((2048, D), lambda i: (i, 0)),
    out_shape=jax.ShapeDtypeStruct((N, D), jnp.float32),
)


def kernel(adj_indices, adj_values, embs):
    rows = adj_indices[0]
    cols = adj_indices[1]
    pad = NNZ_PAD - NNZ
    zi = jnp.zeros((pad,), jnp.int32)
    rows_p = jnp.concatenate([rows, zi]).reshape(NNZ_PAD // 128, 128)
    cols_p = jnp.concatenate([cols, zi]).reshape(NNZ_PAD // 128, 128)
    vals_p = jnp.concatenate([adj_values, jnp.zeros((pad,), jnp.float32)])

    e_h = [embs[:, h * DH:(h + 1) * DH] for h in range(2)]
    y1_h = []
    for h in range(2):
        p = _propagate(rows_p, cols_p, vals_p, e_h[h]).reshape(2, N, DH)
        y1_h.append(_combine_add(p))
    q = [_propagate(cols_p, rows_p, vals_p, y1_h[h]).reshape(2, N, DH)
         for h in range(2)]
    return _combine_act(q[0], q[1])


# speedup vs baseline: 22.1732x; 1.4700x over previous
import jax
import jax.numpy as jnp
from jax import lax
from jax.experimental import pallas as pl
from jax.experimental.pallas import tpu as pltpu
from jax.experimental.pallas import tpu_sc as plsc

N = 16384
D = 64
DH = 32
NNZ = 2684354
LEAKY = 0.2

NW = 32
CHUNK = 1024
KSTREAM = CHUNK // 128
CHUNKS_PER_W = -(-NNZ // (NW * CHUNK))
NNZ_PAD = NW * CHUNK * CHUNKS_PER_W
ROWS_PER_SUB = N // 16


def _propagate_body(src_hbm, dst_hbm, vals_hbm, table_hbm, out_hbm,
                    src_v, dst_v, vals_v, rowbuf0, rowbuf1, acc,
                    gsem0, gsem1, ssem0, ssem1, isem0, isem1):
    cid = lax.axis_index("c")
    sid = lax.axis_index("s")
    wid = sid * 2 + cid
    chunk0 = wid * CHUNKS_PER_W

    rowbufs = (rowbuf0, rowbuf1)
    gsems = (gsem0, gsem1)
    ssems = (ssem0, ssem1)
    isems = (isem0, isem1)

    def fire_idx(g, isem):
        m = lax.rem(g, 3)
        row0 = (chunk0 + g) * KSTREAM
        pltpu.async_copy(src_hbm.at[pl.ds(row0, KSTREAM)], src_v.at[m], isem)
        pltpu.async_copy(dst_hbm.at[pl.ds(row0, KSTREAM)], dst_v.at[m], isem)
        pltpu.async_copy(vals_hbm.at[pl.ds(row0 * 128, CHUNK)],
                         vals_v.at[m], isem)

    def wait_idx(isem):
        pltpu.make_async_copy(src_hbm.at[pl.ds(0, KSTREAM)],
                              src_v.at[0], isem).wait()
        pltpu.make_async_copy(dst_hbm.at[pl.ds(0, KSTREAM)],
                              dst_v.at[0], isem).wait()
        pltpu.make_async_copy(vals_hbm.at[pl.ds(0, CHUNK)],
                              vals_v.at[0], isem).wait()

    def fire_gather(g, rb, gsem):
        m = lax.rem(g, 3)

        @pl.loop(0, KSTREAM)
        def _fg(j):
            pltpu.async_copy(table_hbm.at[src_v.at[m].at[j]],
                             rb.at[pl.ds(j * 128, 128)], gsem)

    def wait_gather(rb, gsem):
        pltpu.make_async_copy(table_hbm.at[pl.ds(0, CHUNK)], rb, gsem).wait()

    def fire_scatter(g, rb, ssem):
        m = lax.rem(g, 3)

        @pl.loop(0, KSTREAM)
        def _fs(j):
            pltpu.async_copy(rb.at[pl.ds(j * 128, 128)],
                             acc.at[dst_v.at[m].at[j]], ssem, add=True)

    def wait_scatter(rb, ssem):
        pltpu.make_async_copy(rb, acc.at[pl.ds(0, CHUNK)], ssem).wait()

    def scale(g, rb):
        m = lax.rem(g, 3)

        @pl.loop(0, CHUNK // 16)
        def _sc(i16):
            vv = vals_v[m, pl.ds(i16 * 16, 16)]
            for lane in range(16):
                v = vv[lane]
                e = i16 * 16 + lane
                for k in range(DH // 16):
                    sl = pl.ds(k * 16, 16)
                    rb[e, sl] = rb[e, sl] * v

    @pl.loop(0, ROWS_PER_SUB)
    def _zero(i):
        for k in range(DH // 16):
            rowbuf0[i, pl.ds(k * 16, 16)] = jnp.zeros((16,), jnp.float32)

    pltpu.sync_copy(rowbuf0, acc.at[pl.ds(sid * ROWS_PER_SUB, ROWS_PER_SUB)])
    plsc.subcore_barrier()

    fire_idx(jnp.int32(0), isems[0])
    fire_idx(jnp.int32(1), isems[1])
    wait_idx(isems[0])
    fire_gather(jnp.int32(0), rowbufs[0], gsems[0])

    @pl.loop(0, CHUNKS_PER_W // 2)
    def _outer(gg):
        for half in range(2):
            g = gg * 2 + half
            p, q = half, 1 - half
            wait_gather(rowbufs[p], gsems[p])
            scale(g, rowbufs[p])
            fire_scatter(g, rowbufs[p], ssems[p])

            @pl.when(g + 1 < CHUNKS_PER_W)
            def _widx():
                wait_idx(isems[q])

            @pl.when(g >= 1)
            def _wsc():
                wait_scatter(rowbufs[q], ssems[q])

            @pl.when(g + 1 < CHUNKS_PER_W)
            def _fg():
                fire_gather(g + 1, rowbufs[q], gsems[q])

            @pl.when(g + 2 < CHUNKS_PER_W)
            def _fi():
                fire_idx(g + 2, isems[p])

    wait_scatter(rowbufs[1], ssems[1])
    plsc.subcore_barrier()
    pltpu.sync_copy(acc.at[pl.ds(sid * ROWS_PER_SUB, ROWS_PER_SUB)],
                    out_hbm.at[pl.ds(cid * N + sid * ROWS_PER_SUB,
                                     ROWS_PER_SUB)])


_propagate = pl.kernel(
    _propagate_body,
    out_type=jax.ShapeDtypeStruct((2 * N, DH), jnp.float32),
    mesh=plsc.VectorSubcoreMesh(core_axis_name="c", subcore_axis_name="s"),
    scratch_types=[
        pltpu.VMEM((3, KSTREAM, 128), jnp.int32),
        pltpu.VMEM((3, KSTREAM, 128), jnp.int32),
        pltpu.VMEM((3, CHUNK), jnp.float32),
        pltpu.VMEM((CHUNK, DH), jnp.float32),
        pltpu.VMEM((CHUNK, DH), jnp.float32),
        pltpu.VMEM_SHARED((N, DH), jnp.float32),
        pltpu.SemaphoreType.DMA,
        pltpu.SemaphoreType.DMA,
        pltpu.SemaphoreType.DMA,
        pltpu.SemaphoreType.DMA,
        pltpu.SemaphoreType.DMA,
        pltpu.SemaphoreType.DMA,
    ],
    compiler_params=pltpu.CompilerParams(use_tc_tiling_on_sc=False),
)


def _combine_add_body(p_ref, o_ref):
    o_ref[...] = p_ref[0] + p_ref[1]


_combine_add = pl.pallas_call(
    _combine_add_body,
    grid=(N // 2048,),
    in_specs=[pl.BlockSpec((2, 2048, DH), lambda i: (0, i, 0))],
    out_specs=pl.BlockSpec((2048, DH), lambda i: (i, 0)),
    out_shape=jax.ShapeDtypeStruct((N, DH), jnp.float32),
)


def _combine_act_body(h0_ref, h1_ref, o_ref):
    z0 = h0_ref[0] + h0_ref[1]
    z1 = h1_ref[0] + h1_ref[1]
    z = jnp.concatenate([z0, z1], axis=1)
    o_ref[...] = jnp.where(z >= 0, z, LEAKY * z)


_combine_act = pl.pallas_call(
    _combine_act_body,
    grid=(N // 2048,),
    in_specs=[pl.BlockSpec((2, 2048, DH), lambda i: (0, i, 0)),
              pl.BlockSpec((2, 2048, DH), lambda i: (0, i, 0))],
    out_specs=pl.BlockSpec((2048, D), lambda i: (i, 0)),
    out_shape=jax.ShapeDtypeStruct((N, D), jnp.float32),
)


def kernel(adj_indices, adj_values, embs):
    rows = adj_indices[0]
    cols = adj_indices[1]
    pad = NNZ_PAD - NNZ
    zi = jnp.zeros((pad,), jnp.int32)
    rows_p = jnp.concatenate([rows, zi]).reshape(NNZ_PAD // 128, 128)
    cols_p = jnp.concatenate([cols, zi]).reshape(NNZ_PAD // 128, 128)
    vals_p = jnp.concatenate([adj_values, jnp.zeros((pad,), jnp.float32)])

    e_h = [embs[:, h * DH:(h + 1) * DH] for h in range(2)]
    y1_h = []
    for h in range(2):
        p = _propagate(rows_p, cols_p, vals_p, e_h[h]).reshape(2, N, DH)
        y1_h.append(_combine_add(p))
    q = [_propagate(cols_p, rows_p, vals_p, y1_h[h]).reshape(2, N, DH)
         for h in range(2)]
    return _combine_act(q[0], q[1])
